# Initial kernel scaffold; baseline (speedup 1.0000x reference)
import jax
import jax.numpy as jnp
from jax.experimental import pallas as pl


def kernel(x, edge_index, edge_attr, batch, params):
    raise NotImplementedError("write your pallas kernel here")


# speedup vs baseline: 2.1123x; 2.1123x over previous
import functools

import jax
import jax.numpy as jnp
from jax import lax
from jax.experimental import pallas as pl
from jax.experimental.pallas import tpu as pltpu
from jax.experimental.pallas import tpu_sc as plsc

H = 128
HEADS = 4
NUM_GRAPHS = 64
EPS = 1e-5
NC = 2
NS = 16
NW = NC * NS
CHUNK = 128
BE = 3200


def _sc_mesh():
    return plsc.VectorSubcoreMesh(core_axis_name="c", subcore_axis_name="s")


def _make_gather(E, N):
    nch = E // CHUNK
    maxit = (nch + NW - 1) // NW

    @functools.partial(
        pl.kernel,
        mesh=_sc_mesh(),
        out_type=(
            jax.ShapeDtypeStruct((E, H), jnp.float32),
            jax.ShapeDtypeStruct((E, H), jnp.float32),
        ),
        scratch_types=[
            pltpu.VMEM((CHUNK,), jnp.int32),
            pltpu.VMEM((CHUNK,), jnp.int32),
            pltpu.VMEM((CHUNK, H), jnp.float32),
            pltpu.VMEM((CHUNK, H), jnp.float32),
            pltpu.SemaphoreType.DMA,
        ],
    )
    def gather(h_hbm, dst_hbm, src_hbm, hd_hbm, hs_hbm,
               idxd_v, idxs_v, rowd_v, rows_v, sem):
        wid = lax.axis_index("s") * NC + lax.axis_index("c")

        def body(it, carry):
            chunk = wid + it * NW

            @pl.when(chunk < nch)
            def _():
                base = chunk * CHUNK
                pltpu.sync_copy(dst_hbm.at[pl.ds(base, CHUNK)], idxd_v)
                pltpu.sync_copy(src_hbm.at[pl.ds(base, CHUNK)], idxs_v)
                cp1 = pltpu.async_copy(h_hbm.at[idxd_v], rowd_v, sem)
                cp2 = pltpu.async_copy(h_hbm.at[idxs_v], rows_v, sem)
                cp1.wait()
                cp2.wait()
                pltpu.sync_copy(rowd_v, hd_hbm.at[pl.ds(base, CHUNK)])
                pltpu.sync_copy(rows_v, hs_hbm.at[pl.ds(base, CHUNK)])

            return carry

        lax.fori_loop(0, maxit, body, 0)

    return gather


def _make_scatter(E, N):
    nch = E // CHUNK
    maxit = (nch + NW - 1) // NW
    D = H

    @functools.partial(
        pl.kernel,
        mesh=_sc_mesh(),
        out_type=jax.ShapeDtypeStruct((NC, N, D), jnp.float32),
        scratch_types=[
            pltpu.VMEM((CHUNK,), jnp.int32),
            pltpu.VMEM((CHUNK, D), jnp.float32),
            pltpu.VMEM_SHARED((N, D), jnp.float32),
        ],
    )
    def scatter(wm_hbm, dst_hbm, zeros_hbm, out_hbm, idx_v, rows_v, acc_sh):
        c = lax.axis_index("c")
        s = lax.axis_index("s")
        wid = s * NC + c

        @pl.when(s == 0)
        def _():
            pltpu.sync_copy(zeros_hbm, acc_sh)

        plsc.subcore_barrier()

        def body(it, carry):
            chunk = wid + it * NW

            @pl.when(chunk < nch)
            def _():
                base = chunk * CHUNK
                pltpu.sync_copy(dst_hbm.at[pl.ds(base, CHUNK)], idx_v)
                pltpu.sync_copy(wm_hbm.at[pl.ds(base, CHUNK)], rows_v)
                pltpu.sync_copy(rows_v, acc_sh.at[idx_v], add=True)

            return carry

        lax.fori_loop(0, maxit, body, 0)
        plsc.subcore_barrier()

        @pl.when(s == 0)
        def _():
            pltpu.sync_copy(acc_sh, out_hbm.at[c])

    return scatter


def _prologue_body(x_ref, w_ref, b_ref, h_ref):
    h_ref[...] = jnp.maximum(
        jnp.dot(x_ref[...], w_ref[...], preferred_element_type=jnp.float32)
        + b_ref[...], 0.0)


def _edge_body(hd_ref, hs_ref, ea_ref, wee_ref, eeb_ref, wd_ref, ws_ref,
               we_ref, bcat_ref, a2_ref, a2b_ref, rm_ref, lt_ref):
    e = jnp.maximum(
        jnp.dot(ea_ref[...], wee_ref[...], preferred_element_type=jnp.float32)
        + eeb_ref[...], 0.0)
    p = (jnp.dot(hd_ref[...], wd_ref[...], preferred_element_type=jnp.float32)
         + jnp.dot(hs_ref[...], ws_ref[...], preferred_element_type=jnp.float32)
         + jnp.dot(e, we_ref[...], preferred_element_type=jnp.float32)
         + bcat_ref[...])
    a = p[:, :H]
    la = jnp.where(a > 0, a, 0.2 * a)
    rm_ref[...] = jnp.maximum(p[:, H:], 0.0)
    lt_ref[...] = lax.dot_general(
        a2_ref[...], la, (((0,), (1,)), ((), ())),
        preferred_element_type=jnp.float32) + a2b_ref[...]


def _softmax_body(lt_ref, w_ref):
    l = lt_ref[...]
    m = jnp.max(l, axis=1, keepdims=True)
    p = jnp.exp(l - m)
    s = jnp.sum(p, axis=1, keepdims=True)
    p = p / s
    w_ref[...] = jnp.sum(p[:HEADS, :], axis=0, keepdims=True) * (1.0 / HEADS)


def _scale_body(rm_ref, w_ref, m2_ref, m2b_ref, wm_ref):
    wv = w_ref[...]
    msg = jnp.dot(rm_ref[...], m2_ref[...],
                  preferred_element_type=jnp.float32) + m2b_ref[...]
    wm_ref[...] = msg * wv


def _node_body(parts_ref, h_ref, u1h_ref, u1a_ref, u1b_ref,
               u2_ref, u2b_ref, gnw_ref, gnb_ref, gnms_ref, lnw_ref,
               lnb_ref, hout_ref):
    agg = parts_ref[0] + parts_ref[1]
    h = h_ref[...]
    z = jnp.maximum(
        jnp.dot(h, u1h_ref[...], preferred_element_type=jnp.float32)
        + jnp.dot(agg, u1a_ref[...], preferred_element_type=jnp.float32)
        + u1b_ref[...], 0.0)
    out = jnp.dot(z, u2_ref[...], preferred_element_type=jnp.float32) + u2b_ref[...]
    mean = jnp.mean(out, axis=0, keepdims=True)
    sub = out - gnms_ref[...] * mean
    var = jnp.mean(sub * sub, axis=0, keepdims=True)
    out = gnw_ref[...] * sub * lax.rsqrt(var + EPS) + gnb_ref[...]
    hres = out + h
    mu = jnp.mean(hres, axis=1, keepdims=True)
    d = hres - mu
    v = jnp.mean(d * d, axis=1, keepdims=True)
    hout_ref[...] = lnw_ref[...] * d * lax.rsqrt(v + EPS) + lnb_ref[...]


def _pool_body(h_ref, b_ref, w1_ref, b1_ref, w2_ref, b2_ref, w3_ref, b3_ref,
               out_ref, hmax_ref):
    h = h_ref[...]
    n = h.shape[0]
    b = b_ref[...]
    gids = lax.broadcasted_iota(jnp.int32, (n, NUM_GRAPHS), 1)
    onehot = (b == gids).astype(jnp.float32)
    counts = lax.dot_general(onehot, jnp.ones((n, 1), jnp.float32),
                             (((0,), (0,)), ((), ())),
                             preferred_element_type=jnp.float32)
    h_add = lax.dot_general(onehot, h, (((0,), (0,)), ((), ())),
                            preferred_element_type=jnp.float32)
    h_mean = h_add / jnp.maximum(counts, 1.0)

    def mx(g, carry):
        mask = b == g
        mv = jnp.max(jnp.where(mask, h, -jnp.inf), axis=0, keepdims=True)
        hmax_ref[pl.ds(g, 1), :] = mv
        return carry

    lax.fori_loop(0, NUM_GRAPHS, mx, 0)
    h_max = hmax_ref[...]
    h_max = jnp.where(jnp.isfinite(h_max), h_max, 0.0)
    hg = jnp.concatenate([h_mean, h_max, h_add], axis=1)

    cols = []
    for k in range(3):
        z1 = jnp.maximum(
            jnp.dot(hg, w1_ref[pl.ds(k * 384, 384), :],
                    preferred_element_type=jnp.float32)
            + b1_ref[pl.ds(k, 1), :], 0.0)
        z2 = jnp.maximum(
            jnp.dot(z1, w2_ref[pl.ds(k * H, H), :],
                    preferred_element_type=jnp.float32)
            + b2_ref[pl.ds(k, 1), :], 0.0)
        z3 = (jnp.dot(z2, w3_ref[pl.ds(k * 64, 64), :],
                      preferred_element_type=jnp.float32)
              + b3_ref[pl.ds(k, 1), :])
        cols.append(jax.nn.sigmoid(z3[:, :1]))
    comb = 0.45 * cols[0] + 0.35 * cols[1] + 0.2 * cols[2]
    out_ref[...] = jnp.concatenate(cols + [comb], axis=1)


def kernel(x, edge_index, edge_attr, batch, params):
    N = x.shape[0]
    E = edge_index.shape[1]
    f32 = jnp.float32
    src = edge_index[0]
    dst = edge_index[1]
    ea8 = jnp.concatenate(
        [edge_attr, jnp.zeros((E, 1), f32)], axis=1)
    wee8 = jnp.concatenate(
        [params['ee'][0], jnp.zeros((1, H), f32)], axis=0)
    eeb = params['ee'][1][None, :]

    h = pl.pallas_call(
        _prologue_body,
        out_shape=jax.ShapeDtypeStruct((N, H), f32),
    )(x, params['ne'][0], params['ne'][1][None, :])

    gather = _make_gather(E, N)
    scatter = _make_scatter(E, N)
    nblk = E // BE
    zeros_acc = jnp.zeros((N, H), f32)

    for lp in params['layers']:
        a1w, a1b = lp['a1']
        m1w, m1b = lp['m1']
        wd = jnp.concatenate([a1w[:H], m1w[:H]], axis=1)
        ws = jnp.concatenate([a1w[H:2 * H], m1w[H:2 * H]], axis=1)
        we = jnp.concatenate([a1w[2 * H:], m1w[2 * H:]], axis=1)
        bcat = jnp.concatenate([a1b, m1b])[None, :]
        a2p = jnp.concatenate(
            [lp['a2'][0], jnp.zeros((H, 8 - HEADS), f32)], axis=1)
        a2bp = jnp.concatenate(
            [lp['a2'][1], jnp.zeros((8 - HEADS,), f32)])[:, None]
        m2w, m2b = lp['m2']
        u1w, u1b = lp['u1']

        hd, hs = gather(h, dst, src)

        rm, lt = pl.pallas_call(
            _edge_body,
            grid=(nblk,),
            in_specs=[
                pl.BlockSpec((BE, H), lambda i: (i, 0)),
                pl.BlockSpec((BE, H), lambda i: (i, 0)),
                pl.BlockSpec((BE, 8), lambda i: (i, 0)),
                pl.BlockSpec((8, H), lambda i: (0, 0)),
                pl.BlockSpec((1, H), lambda i: (0, 0)),
                pl.BlockSpec((H, 2 * H), lambda i: (0, 0)),
                pl.BlockSpec((H, 2 * H), lambda i: (0, 0)),
                pl.BlockSpec((H, 2 * H), lambda i: (0, 0)),
                pl.BlockSpec((1, 2 * H), lambda i: (0, 0)),
                pl.BlockSpec((H, 8), lambda i: (0, 0)),
                pl.BlockSpec((8, 1), lambda i: (0, 0)),
            ],
            out_specs=[
                pl.BlockSpec((BE, H), lambda i: (i, 0)),
                pl.BlockSpec((8, BE), lambda i: (0, i)),
            ],
            out_shape=[
                jax.ShapeDtypeStruct((E, H), f32),
                jax.ShapeDtypeStruct((8, E), f32),
            ],
        )(hd, hs, ea8, wee8, eeb, wd, ws, we, bcat, a2p, a2bp)

        w = pl.pallas_call(
            _softmax_body,
            out_shape=jax.ShapeDtypeStruct((1, E), f32),
        )(lt)
        w = w.reshape(E, 1)

        wm = pl.pallas_call(
            _scale_body,
            grid=(nblk,),
            in_specs=[
                pl.BlockSpec((BE, H), lambda i: (i, 0)),
                pl.BlockSpec((BE, 1), lambda i: (i, 0)),
                pl.BlockSpec((H, H), lambda i: (0, 0)),
                pl.BlockSpec((1, H), lambda i: (0, 0)),
            ],
            out_specs=pl.BlockSpec((BE, H), lambda i: (i, 0)),
            out_shape=jax.ShapeDtypeStruct((E, H), f32),
        )(rm, w, m2w, m2b[None, :])

        parts = scatter(wm, dst, zeros_acc)

        h = pl.pallas_call(
            _node_body,
            out_shape=jax.ShapeDtypeStruct((N, H), f32),
        )(parts, h, u1w[:H], u1w[H:], u1b[None, :],
          lp['u2'][0], lp['u2'][1][None, :],
          lp['gn_w'][None, :], lp['gn_b'][None, :], lp['gn_ms'][None, :],
          lp['ln_w'][None, :], lp['ln_b'][None, :])

    hp = params['heads']
    w1 = jnp.concatenate([hp[k]['h1'][0] for k in ('tensile', 'tg', 'flex')], 0)
    b1 = jnp.stack([hp[k]['h1'][1] for k in ('tensile', 'tg', 'flex')], 0)
    w2 = jnp.concatenate([hp[k]['h2'][0] for k in ('tensile', 'tg', 'flex')], 0)
    b2 = jnp.stack([hp[k]['h2'][1] for k in ('tensile', 'tg', 'flex')], 0)
    w3 = jnp.concatenate(
        [jnp.concatenate([hp[k]['h3'][0],
                          jnp.zeros((64, 127), f32)], axis=1)
         for k in ('tensile', 'tg', 'flex')], 0)
    b3 = jnp.stack([jnp.concatenate([hp[k]['h3'][1],
                                     jnp.zeros((127,), f32)])
                    for k in ('tensile', 'tg', 'flex')], 0)

    outm = pl.pallas_call(
        _pool_body,
        out_shape=jax.ShapeDtypeStruct((NUM_GRAPHS, 4), f32),
        scratch_shapes=[pltpu.VMEM((NUM_GRAPHS, H), f32)],
    )(h, batch[:, None], w1, b1, w2, b2, w3, b3)

    return (outm[:, 0], outm[:, 1], outm[:, 2], outm[:, 3])


# speedup vs baseline: 2.3445x; 1.1099x over previous
import functools

import jax
import jax.numpy as jnp
from jax import lax
from jax.experimental import pallas as pl
from jax.experimental.pallas import tpu as pltpu
from jax.experimental.pallas import tpu_sc as plsc

H = 128
HEADS = 4
NUM_GRAPHS = 64
EPS = 1e-5
NC = 2
NS = 16
NW = NC * NS
CHUNK = 128
BE = 3200


def _sc_mesh():
    return plsc.VectorSubcoreMesh(core_axis_name="c", subcore_axis_name="s")


def _make_gather(E, N):
    nch = E // CHUNK
    maxit = (nch + NW - 1) // NW

    @functools.partial(
        pl.kernel,
        mesh=_sc_mesh(),
        out_type=(
            jax.ShapeDtypeStruct((E, H), jnp.float32),
            jax.ShapeDtypeStruct((E, H), jnp.float32),
        ),
        scratch_types=[
            pltpu.VMEM((CHUNK,), jnp.int32),
            pltpu.VMEM((CHUNK,), jnp.int32),
            pltpu.VMEM((CHUNK, H), jnp.float32),
            pltpu.VMEM((CHUNK, H), jnp.float32),
            pltpu.SemaphoreType.DMA,
        ],
    )
    def gather(h_hbm, dst_hbm, src_hbm, hd_hbm, hs_hbm,
               idxd_v, idxs_v, rowd_v, rows_v, sem):
        wid = lax.axis_index("s") * NC + lax.axis_index("c")

        def body(it, carry):
            chunk = wid + it * NW

            @pl.when(chunk < nch)
            def _():
                base = chunk * CHUNK
                pltpu.sync_copy(dst_hbm.at[pl.ds(base, CHUNK)], idxd_v)
                pltpu.sync_copy(src_hbm.at[pl.ds(base, CHUNK)], idxs_v)
                cp1 = pltpu.async_copy(h_hbm.at[idxd_v], rowd_v, sem)
                cp2 = pltpu.async_copy(h_hbm.at[idxs_v], rows_v, sem)
                cp1.wait()
                cp2.wait()
                pltpu.sync_copy(rowd_v, hd_hbm.at[pl.ds(base, CHUNK)])
                pltpu.sync_copy(rows_v, hs_hbm.at[pl.ds(base, CHUNK)])

            return carry

        lax.fori_loop(0, maxit, body, 0)

    return gather


def _make_scatter(E, N):
    nch = E // CHUNK
    maxit = (nch + NW - 1) // NW
    D = H

    @functools.partial(
        pl.kernel,
        mesh=_sc_mesh(),
        out_type=jax.ShapeDtypeStruct((NC, N, D), jnp.float32),
        scratch_types=[
            pltpu.VMEM((CHUNK,), jnp.int32),
            pltpu.VMEM((CHUNK, D), jnp.float32),
            pltpu.VMEM_SHARED((N, D), jnp.float32),
        ],
    )
    def scatter(wm_hbm, dst_hbm, zeros_hbm, out_hbm, idx_v, rows_v, acc_sh):
        c = lax.axis_index("c")
        s = lax.axis_index("s")
        wid = s * NC + c

        @pl.when(s == 0)
        def _():
            pltpu.sync_copy(zeros_hbm, acc_sh)

        plsc.subcore_barrier()

        def body(it, carry):
            chunk = wid + it * NW

            @pl.when(chunk < nch)
            def _():
                base = chunk * CHUNK
                pltpu.sync_copy(dst_hbm.at[pl.ds(base, CHUNK)], idx_v)
                pltpu.sync_copy(wm_hbm.at[pl.ds(base, CHUNK)], rows_v)
                pltpu.sync_copy(rows_v, acc_sh.at[idx_v], add=True)

            return carry

        lax.fori_loop(0, maxit, body, 0)
        plsc.subcore_barrier()

        @pl.when(s == 0)
        def _():
            pltpu.sync_copy(acc_sh, out_hbm.at[c])

    return scatter


def _prologue_body(x_ref, w_ref, b_ref, h_ref):
    h_ref[...] = jnp.maximum(
        jnp.dot(x_ref[...], w_ref[...], preferred_element_type=jnp.float32)
        + b_ref[...], 0.0)


def _edge_body(hd_ref, hs_ref, ea_ref, wee_ref, eeb_ref, wd_ref, ws_ref,
               we_ref, bcat_ref, a2_ref, a2b_ref, rm_ref, lt_ref):
    e = jnp.maximum(
        jnp.dot(ea_ref[...], wee_ref[...], preferred_element_type=jnp.float32)
        + eeb_ref[...], 0.0)
    p = (jnp.dot(hd_ref[...], wd_ref[...], preferred_element_type=jnp.float32)
         + jnp.dot(hs_ref[...], ws_ref[...], preferred_element_type=jnp.float32)
         + jnp.dot(e, we_ref[...], preferred_element_type=jnp.float32)
         + bcat_ref[...])
    a = p[:, :H]
    la = jnp.where(a > 0, a, 0.2 * a)
    rm_ref[...] = jnp.maximum(p[:, H:], 0.0)
    lt_ref[...] = lax.dot_general(
        a2_ref[...], la, (((0,), (1,)), ((), ())),
        preferred_element_type=jnp.float32) + a2b_ref[...]


def _softmax_body(lt0_ref, lt1_ref, w_ref):
    l0 = lt0_ref[...]
    l1 = lt1_ref[...]
    m = jnp.maximum(jnp.max(l0, axis=1, keepdims=True),
                    jnp.max(l1, axis=1, keepdims=True))
    p0 = jnp.exp(l0 - m)
    p1 = jnp.exp(l1 - m)
    s = (jnp.sum(p0, axis=1, keepdims=True)
         + jnp.sum(p1, axis=1, keepdims=True))
    w0 = jnp.sum(p0[:HEADS, :] / s[:HEADS], axis=0, keepdims=True)
    w1 = jnp.sum(p1[:HEADS, :] / s[:HEADS], axis=0, keepdims=True)
    w_ref[...] = jnp.concatenate([w0, w1], axis=1) * (1.0 / HEADS)


def _scale_body(rm_ref, w_ref, m2_ref, m2b_ref, wm_ref):
    wv = w_ref[...]
    msg = jnp.dot(rm_ref[...], m2_ref[...],
                  preferred_element_type=jnp.float32) + m2b_ref[...]
    wm_ref[...] = msg * wv


def _node_body(parts0_ref, parts1_ref, h_ref, u1h_ref, u1a_ref, u1b_ref,
               u2_ref, u2b_ref, gnw_ref, gnb_ref, gnms_ref, lnw_ref,
               lnb_ref, hout_ref):
    agg = (parts0_ref[0] + parts0_ref[1]
           + parts1_ref[0] + parts1_ref[1])
    h = h_ref[...]
    z = jnp.maximum(
        jnp.dot(h, u1h_ref[...], preferred_element_type=jnp.float32)
        + jnp.dot(agg, u1a_ref[...], preferred_element_type=jnp.float32)
        + u1b_ref[...], 0.0)
    out = jnp.dot(z, u2_ref[...], preferred_element_type=jnp.float32) + u2b_ref[...]
    mean = jnp.mean(out, axis=0, keepdims=True)
    sub = out - gnms_ref[...] * mean
    var = jnp.mean(sub * sub, axis=0, keepdims=True)
    out = gnw_ref[...] * sub * lax.rsqrt(var + EPS) + gnb_ref[...]
    hres = out + h
    mu = jnp.mean(hres, axis=1, keepdims=True)
    d = hres - mu
    v = jnp.mean(d * d, axis=1, keepdims=True)
    hout_ref[...] = lnw_ref[...] * d * lax.rsqrt(v + EPS) + lnb_ref[...]


def _pool_body(h_ref, b_ref, w1_ref, b1_ref, w2_ref, b2_ref, w3_ref, b3_ref,
               out_ref, hmax_ref):
    h = h_ref[...]
    n = h.shape[0]
    b = b_ref[...]
    gids = lax.broadcasted_iota(jnp.int32, (n, NUM_GRAPHS), 1)
    onehot = (b == gids).astype(jnp.float32)
    counts = lax.dot_general(onehot, jnp.ones((n, 1), jnp.float32),
                             (((0,), (0,)), ((), ())),
                             preferred_element_type=jnp.float32)
    h_add = lax.dot_general(onehot, h, (((0,), (0,)), ((), ())),
                            preferred_element_type=jnp.float32)
    h_mean = h_add / jnp.maximum(counts, 1.0)

    def mx(g, carry):
        mask = b == g
        mv = jnp.max(jnp.where(mask, h, -jnp.inf), axis=0, keepdims=True)
        hmax_ref[pl.ds(g, 1), :] = mv
        return carry

    lax.fori_loop(0, NUM_GRAPHS, mx, 0)
    h_max = hmax_ref[...]
    h_max = jnp.where(jnp.isfinite(h_max), h_max, 0.0)
    hg = jnp.concatenate([h_mean, h_max, h_add], axis=1)

    cols = []
    for k in range(3):
        z1 = jnp.maximum(
            jnp.dot(hg, w1_ref[pl.ds(k * 384, 384), :],
                    preferred_element_type=jnp.float32)
            + b1_ref[pl.ds(k, 1), :], 0.0)
        z2 = jnp.maximum(
            jnp.dot(z1, w2_ref[pl.ds(k * H, H), :],
                    preferred_element_type=jnp.float32)
            + b2_ref[pl.ds(k, 1), :], 0.0)
        z3 = (jnp.dot(z2, w3_ref[pl.ds(k * 64, 64), :],
                      preferred_element_type=jnp.float32)
              + b3_ref[pl.ds(k, 1), :])
        cols.append(jax.nn.sigmoid(z3[:, :1]))
    comb = 0.45 * cols[0] + 0.35 * cols[1] + 0.2 * cols[2]
    out_ref[...] = jnp.concatenate(cols + [comb], axis=1)


def kernel(x, edge_index, edge_attr, batch, params):
    N = x.shape[0]
    E = edge_index.shape[1]
    f32 = jnp.float32
    src = edge_index[0]
    dst = edge_index[1]
    ea8 = jnp.concatenate(
        [edge_attr, jnp.zeros((E, 1), f32)], axis=1)
    wee8 = jnp.concatenate(
        [params['ee'][0], jnp.zeros((1, H), f32)], axis=0)
    eeb = params['ee'][1][None, :]

    h = pl.pallas_call(
        _prologue_body,
        out_shape=jax.ShapeDtypeStruct((N, H), f32),
    )(x, params['ne'][0], params['ne'][1][None, :])

    ES = E // 2
    slabs = [(dst[:ES], src[:ES], ea8[:ES]), (dst[ES:], src[ES:], ea8[ES:])]
    gather = _make_gather(ES, N)
    scatter = _make_scatter(ES, N)
    nblk = ES // BE
    zeros_acc = jnp.zeros((N, H), f32)

    for lp in params['layers']:
        a1w, a1b = lp['a1']
        m1w, m1b = lp['m1']
        wd = jnp.concatenate([a1w[:H], m1w[:H]], axis=1)
        ws = jnp.concatenate([a1w[H:2 * H], m1w[H:2 * H]], axis=1)
        we = jnp.concatenate([a1w[2 * H:], m1w[2 * H:]], axis=1)
        bcat = jnp.concatenate([a1b, m1b])[None, :]
        a2p = jnp.concatenate(
            [lp['a2'][0], jnp.zeros((H, 8 - HEADS), f32)], axis=1)
        a2bp = jnp.concatenate(
            [lp['a2'][1], jnp.zeros((8 - HEADS,), f32)])[:, None]
        m2w, m2b = lp['m2']
        u1w, u1b = lp['u1']

        gathered = [gather(h, d, s) for d, s, _ in slabs]

        rms, lts = [], []
        for (hd, hs), (_, _, eas) in zip(gathered, slabs):
            rm, lt = pl.pallas_call(
                _edge_body,
                grid=(nblk,),
                in_specs=[
                    pl.BlockSpec((BE, H), lambda i: (i, 0)),
                    pl.BlockSpec((BE, H), lambda i: (i, 0)),
                    pl.BlockSpec((BE, 8), lambda i: (i, 0)),
                    pl.BlockSpec((8, H), lambda i: (0, 0)),
                    pl.BlockSpec((1, H), lambda i: (0, 0)),
                    pl.BlockSpec((H, 2 * H), lambda i: (0, 0)),
                    pl.BlockSpec((H, 2 * H), lambda i: (0, 0)),
                    pl.BlockSpec((H, 2 * H), lambda i: (0, 0)),
                    pl.BlockSpec((1, 2 * H), lambda i: (0, 0)),
                    pl.BlockSpec((H, 8), lambda i: (0, 0)),
                    pl.BlockSpec((8, 1), lambda i: (0, 0)),
                ],
                out_specs=[
                    pl.BlockSpec((BE, H), lambda i: (i, 0)),
                    pl.BlockSpec((8, BE), lambda i: (0, i)),
                ],
                out_shape=[
                    jax.ShapeDtypeStruct((ES, H), f32),
                    jax.ShapeDtypeStruct((8, ES), f32),
                ],
            )(hd, hs, eas, wee8, eeb, wd, ws, we, bcat, a2p, a2bp)
            rms.append(rm)
            lts.append(lt)

        w = pl.pallas_call(
            _softmax_body,
            out_shape=jax.ShapeDtypeStruct((1, E), f32),
        )(lts[0], lts[1])
        w = w.reshape(E, 1)

        parts = []
        for k, ((d, _, _), rm) in enumerate(zip(slabs, rms)):
            wm = pl.pallas_call(
                _scale_body,
                grid=(nblk,),
                in_specs=[
                    pl.BlockSpec((BE, H), lambda i: (i, 0)),
                    pl.BlockSpec((BE, 1), lambda i: (i, 0)),
                    pl.BlockSpec((H, H), lambda i: (0, 0)),
                    pl.BlockSpec((1, H), lambda i: (0, 0)),
                ],
                out_specs=pl.BlockSpec((BE, H), lambda i: (i, 0)),
                out_shape=jax.ShapeDtypeStruct((ES, H), f32),
            )(rm, w[k * ES:(k + 1) * ES], m2w, m2b[None, :])
            parts.append(scatter(wm, d, zeros_acc))

        h = pl.pallas_call(
            _node_body,
            out_shape=jax.ShapeDtypeStruct((N, H), f32),
        )(parts[0], parts[1], h, u1w[:H], u1w[H:], u1b[None, :],
          lp['u2'][0], lp['u2'][1][None, :],
          lp['gn_w'][None, :], lp['gn_b'][None, :], lp['gn_ms'][None, :],
          lp['ln_w'][None, :], lp['ln_b'][None, :])

    hp = params['heads']
    w1 = jnp.concatenate([hp[k]['h1'][0] for k in ('tensile', 'tg', 'flex')], 0)
    b1 = jnp.stack([hp[k]['h1'][1] for k in ('tensile', 'tg', 'flex')], 0)
    w2 = jnp.concatenate([hp[k]['h2'][0] for k in ('tensile', 'tg', 'flex')], 0)
    b2 = jnp.stack([hp[k]['h2'][1] for k in ('tensile', 'tg', 'flex')], 0)
    w3 = jnp.concatenate(
        [jnp.concatenate([hp[k]['h3'][0],
                          jnp.zeros((64, 127), f32)], axis=1)
         for k in ('tensile', 'tg', 'flex')], 0)
    b3 = jnp.stack([jnp.concatenate([hp[k]['h3'][1],
                                     jnp.zeros((127,), f32)])
                    for k in ('tensile', 'tg', 'flex')], 0)

    outm = pl.pallas_call(
        _pool_body,
        out_shape=jax.ShapeDtypeStruct((NUM_GRAPHS, 4), f32),
        scratch_shapes=[pltpu.VMEM((NUM_GRAPHS, H), f32)],
    )(h, batch[:, None], w1, b1, w2, b2, w3, b3)

    return (outm[:, 0], outm[:, 1], outm[:, 2], outm[:, 3])


# speedup vs baseline: 2.5661x; 1.0945x over previous
import functools

import jax
import jax.numpy as jnp
from jax import lax
from jax.experimental import pallas as pl
from jax.experimental.pallas import tpu as pltpu
from jax.experimental.pallas import tpu_sc as plsc

H = 128
HEADS = 4
NUM_GRAPHS = 64
EPS = 1e-5
NC = 2
NS = 16
NW = NC * NS
CHUNK = 128
BE = 3200


def _sc_mesh():
    return plsc.VectorSubcoreMesh(core_axis_name="c", subcore_axis_name="s")


def _make_gather(E, N):
    nch = E // CHUNK
    maxit = (nch + NW - 1) // NW
    maxit += maxit % 2
    npair = maxit // 2

    @functools.partial(
        pl.kernel,
        mesh=_sc_mesh(),
        out_type=(
            jax.ShapeDtypeStruct((E, H), jnp.float32),
            jax.ShapeDtypeStruct((E, H), jnp.float32),
        ),
        scratch_types=[
            pltpu.VMEM((CHUNK,), jnp.int32),
            pltpu.VMEM((CHUNK,), jnp.int32),
            pltpu.VMEM((CHUNK,), jnp.int32),
            pltpu.VMEM((CHUNK,), jnp.int32),
            pltpu.VMEM((CHUNK, H), jnp.float32),
            pltpu.VMEM((CHUNK, H), jnp.float32),
            pltpu.VMEM((CHUNK, H), jnp.float32),
            pltpu.VMEM((CHUNK, H), jnp.float32),
            pltpu.SemaphoreType.DMA,
            pltpu.SemaphoreType.DMA,
            pltpu.SemaphoreType.DMA,
            pltpu.SemaphoreType.DMA,
            pltpu.SemaphoreType.DMA,
            pltpu.SemaphoreType.DMA,
        ],
    )
    def gather(h_hbm, dst_hbm, src_hbm, hd_hbm, hs_hbm,
               idxd0, idxs0, idxd1, idxs1, rowd0, rows0, rowd1, rows1,
               si0, si1, sg0, sg1, sw0, sw1):
        wid = lax.axis_index("s") * NC + lax.axis_index("c")
        idx = ((idxd0, idxs0), (idxd1, idxs1))
        row = ((rowd0, rows0), (rowd1, rows1))
        si = (si0, si1)
        sg = (sg0, sg1)
        sw = (sw0, sw1)

        def base_of(i):
            return jnp.minimum(wid + i * NW, nch - 1) * CHUNK

        b0 = base_of(0)
        pltpu.async_copy(dst_hbm.at[pl.ds(b0, CHUNK)], idxd0, si0)
        pltpu.async_copy(src_hbm.at[pl.ds(b0, CHUNK)], idxs0, si0)

        def body(j, carry):
            for b in (0, 1):
                i = 2 * j + b
                bn = 1 - b
                base_i = base_of(i)
                base_n = base_of(i + 1)
                pltpu.async_copy(dst_hbm.at[pl.ds(base_n, CHUNK)], idx[bn][0],
                                 si[bn])
                pltpu.async_copy(src_hbm.at[pl.ds(base_n, CHUNK)], idx[bn][1],
                                 si[bn])
                pltpu.make_async_copy(dst_hbm.at[pl.ds(base_i, CHUNK)],
                                      idx[b][0], si[b]).wait()
                pltpu.make_async_copy(src_hbm.at[pl.ds(base_i, CHUNK)],
                                      idx[b][1], si[b]).wait()

                @pl.when(j >= 1)
                def _():
                    base_p = base_of(i - 2)
                    pltpu.make_async_copy(row[b][0],
                                          hd_hbm.at[pl.ds(base_p, CHUNK)],
                                          sw[b]).wait()
                    pltpu.make_async_copy(row[b][1],
                                          hs_hbm.at[pl.ds(base_p, CHUNK)],
                                          sw[b]).wait()

                cp1 = pltpu.async_copy(h_hbm.at[idx[b][0]], row[b][0], sg[b])
                cp2 = pltpu.async_copy(h_hbm.at[idx[b][1]], row[b][1], sg[b])
                cp1.wait()
                cp2.wait()
                pltpu.async_copy(row[b][0], hd_hbm.at[pl.ds(base_i, CHUNK)],
                                 sw[b])
                pltpu.async_copy(row[b][1], hs_hbm.at[pl.ds(base_i, CHUNK)],
                                 sw[b])
            return carry

        lax.fori_loop(0, npair, body, 0)
        base_e = base_of(maxit)
        pltpu.make_async_copy(dst_hbm.at[pl.ds(base_e, CHUNK)], idxd0,
                              si0).wait()
        pltpu.make_async_copy(src_hbm.at[pl.ds(base_e, CHUNK)], idxs0,
                              si0).wait()
        for b in (0, 1):
            base_l = base_of(maxit - 2 + b)
            pltpu.make_async_copy(row[b][0], hd_hbm.at[pl.ds(base_l, CHUNK)],
                                  sw[b]).wait()
            pltpu.make_async_copy(row[b][1], hs_hbm.at[pl.ds(base_l, CHUNK)],
                                  sw[b]).wait()

    return gather


def _make_scatter(E, N):
    nch = E // CHUNK
    maxit = (nch + NW - 1) // NW
    maxit += maxit % 2
    npair = maxit // 2
    D = H

    @functools.partial(
        pl.kernel,
        mesh=_sc_mesh(),
        out_type=jax.ShapeDtypeStruct((NC, N, D), jnp.float32),
        scratch_types=[
            pltpu.VMEM((CHUNK,), jnp.int32),
            pltpu.VMEM((CHUNK,), jnp.int32),
            pltpu.VMEM((CHUNK, D), jnp.float32),
            pltpu.VMEM((CHUNK, D), jnp.float32),
            pltpu.VMEM_SHARED((N + 8, D), jnp.float32),
            pltpu.SemaphoreType.DMA,
            pltpu.SemaphoreType.DMA,
        ],
    )
    def scatter(wm_hbm, dst_hbm, zeros_hbm, out_hbm,
                idx0, idx1, rows0, rows1, acc_sh, sl0, sl1):
        c = lax.axis_index("c")
        s = lax.axis_index("s")
        wid = s * NC + c
        idx = (idx0, idx1)
        rows = (rows0, rows1)
        sl = (sl0, sl1)

        @pl.when(s == 0)
        def _():
            pltpu.sync_copy(zeros_hbm, acc_sh)

        plsc.subcore_barrier()

        def base_of(i):
            return jnp.minimum(wid + i * NW, nch - 1) * CHUNK

        b0 = base_of(0)
        pltpu.async_copy(dst_hbm.at[pl.ds(b0, CHUNK)], idx0, sl0)
        pltpu.async_copy(wm_hbm.at[pl.ds(b0, CHUNK)], rows0, sl0)

        def body(j, carry):
            for b in (0, 1):
                i = 2 * j + b
                bn = 1 - b
                base_i = base_of(i)
                base_n = base_of(i + 1)
                pltpu.make_async_copy(dst_hbm.at[pl.ds(base_i, CHUNK)],
                                      idx[b], sl[b]).wait()
                pltpu.make_async_copy(wm_hbm.at[pl.ds(base_i, CHUNK)],
                                      rows[b], sl[b]).wait()

                @pl.when(wid + i * NW >= nch)
                def _():
                    for k in range(CHUNK // 16):
                        idx[b][pl.ds(k * 16, 16)] = jnp.full(
                            (16,), N, jnp.int32)

                pltpu.async_copy(dst_hbm.at[pl.ds(base_n, CHUNK)], idx[bn],
                                 sl[bn])
                pltpu.async_copy(wm_hbm.at[pl.ds(base_n, CHUNK)], rows[bn],
                                 sl[bn])
                pltpu.sync_copy(rows[b], acc_sh.at[idx[b]], add=True)
            return carry

        lax.fori_loop(0, npair, body, 0)
        base_e = base_of(maxit)
        pltpu.make_async_copy(dst_hbm.at[pl.ds(base_e, CHUNK)], idx0,
                              sl0).wait()
        pltpu.make_async_copy(wm_hbm.at[pl.ds(base_e, CHUNK)], rows0,
                              sl0).wait()
        plsc.subcore_barrier()

        @pl.when(s == 0)
        def _():
            pltpu.sync_copy(acc_sh.at[pl.ds(0, N)], out_hbm.at[c])

    return scatter


def _prologue_body(x_ref, w_ref, b_ref, h_ref):
    h_ref[...] = jnp.maximum(
        jnp.dot(x_ref[...], w_ref[...], preferred_element_type=jnp.float32)
        + b_ref[...], 0.0)


def _edge_body(hd_ref, hs_ref, ea_ref, wee_ref, eeb_ref, wd_ref, ws_ref,
               we_ref, bcat_ref, a2_ref, a2b_ref, rm_ref, lt_ref):
    e = jnp.maximum(
        jnp.dot(ea_ref[...], wee_ref[...], preferred_element_type=jnp.float32)
        + eeb_ref[...], 0.0)
    p = (jnp.dot(hd_ref[...], wd_ref[...], preferred_element_type=jnp.float32)
         + jnp.dot(hs_ref[...], ws_ref[...], preferred_element_type=jnp.float32)
         + jnp.dot(e, we_ref[...], preferred_element_type=jnp.float32)
         + bcat_ref[...])
    a = p[:, :H]
    la = jnp.where(a > 0, a, 0.2 * a)
    rm_ref[...] = jnp.maximum(p[:, H:], 0.0)
    lt_ref[...] = lax.dot_general(
        a2_ref[...], la, (((0,), (1,)), ((), ())),
        preferred_element_type=jnp.float32) + a2b_ref[...]


def _softmax_body(lt0_ref, lt1_ref, w_ref):
    l0 = lt0_ref[...]
    l1 = lt1_ref[...]
    m = jnp.maximum(jnp.max(l0, axis=1, keepdims=True),
                    jnp.max(l1, axis=1, keepdims=True))
    p0 = jnp.exp(l0 - m)
    p1 = jnp.exp(l1 - m)
    s = (jnp.sum(p0, axis=1, keepdims=True)
         + jnp.sum(p1, axis=1, keepdims=True))
    w0 = jnp.sum(p0[:HEADS, :] / s[:HEADS], axis=0, keepdims=True)
    w1 = jnp.sum(p1[:HEADS, :] / s[:HEADS], axis=0, keepdims=True)
    w_ref[...] = jnp.concatenate([w0, w1], axis=1) * (1.0 / HEADS)


def _scale_body(rm_ref, w_ref, m2_ref, m2b_ref, wm_ref):
    wv = w_ref[...]
    msg = jnp.dot(rm_ref[...], m2_ref[...],
                  preferred_element_type=jnp.float32) + m2b_ref[...]
    wm_ref[...] = msg * wv


def _node_body(parts0_ref, parts1_ref, h_ref, u1h_ref, u1a_ref, u1b_ref,
               u2_ref, u2b_ref, gnw_ref, gnb_ref, gnms_ref, lnw_ref,
               lnb_ref, hout_ref):
    agg = (parts0_ref[0] + parts0_ref[1]
           + parts1_ref[0] + parts1_ref[1])
    h = h_ref[...]
    z = jnp.maximum(
        jnp.dot(h, u1h_ref[...], preferred_element_type=jnp.float32)
        + jnp.dot(agg, u1a_ref[...], preferred_element_type=jnp.float32)
        + u1b_ref[...], 0.0)
    out = jnp.dot(z, u2_ref[...], preferred_element_type=jnp.float32) + u2b_ref[...]
    mean = jnp.mean(out, axis=0, keepdims=True)
    sub = out - gnms_ref[...] * mean
    var = jnp.mean(sub * sub, axis=0, keepdims=True)
    out = gnw_ref[...] * sub * lax.rsqrt(var + EPS) + gnb_ref[...]
    hres = out + h
    mu = jnp.mean(hres, axis=1, keepdims=True)
    d = hres - mu
    v = jnp.mean(d * d, axis=1, keepdims=True)
    hout_ref[...] = lnw_ref[...] * d * lax.rsqrt(v + EPS) + lnb_ref[...]


def _pool_body(h_ref, b_ref, w1_ref, b1_ref, w2_ref, b2_ref, w3_ref, b3_ref,
               out_ref, hmax_ref):
    h = h_ref[...]
    n = h.shape[0]
    b = b_ref[...]
    gids = lax.broadcasted_iota(jnp.int32, (n, NUM_GRAPHS), 1)
    onehot = (b == gids).astype(jnp.float32)
    counts = lax.dot_general(onehot, jnp.ones((n, 1), jnp.float32),
                             (((0,), (0,)), ((), ())),
                             preferred_element_type=jnp.float32)
    h_add = lax.dot_general(onehot, h, (((0,), (0,)), ((), ())),
                            preferred_element_type=jnp.float32)
    h_mean = h_add / jnp.maximum(counts, 1.0)

    def mx(g, carry):
        mask = b == g
        mv = jnp.max(jnp.where(mask, h, -jnp.inf), axis=0, keepdims=True)
        hmax_ref[pl.ds(g, 1), :] = mv
        return carry

    lax.fori_loop(0, NUM_GRAPHS, mx, 0)
    h_max = hmax_ref[...]
    h_max = jnp.where(jnp.isfinite(h_max), h_max, 0.0)
    hg = jnp.concatenate([h_mean, h_max, h_add], axis=1)

    cols = []
    for k in range(3):
        z1 = jnp.maximum(
            jnp.dot(hg, w1_ref[pl.ds(k * 384, 384), :],
                    preferred_element_type=jnp.float32)
            + b1_ref[pl.ds(k, 1), :], 0.0)
        z2 = jnp.maximum(
            jnp.dot(z1, w2_ref[pl.ds(k * H, H), :],
                    preferred_element_type=jnp.float32)
            + b2_ref[pl.ds(k, 1), :], 0.0)
        z3 = (jnp.dot(z2, w3_ref[pl.ds(k * 64, 64), :],
                      preferred_element_type=jnp.float32)
              + b3_ref[pl.ds(k, 1), :])
        cols.append(jax.nn.sigmoid(z3[:, :1]))
    comb = 0.45 * cols[0] + 0.35 * cols[1] + 0.2 * cols[2]
    out_ref[...] = jnp.concatenate(cols + [comb], axis=1)


def kernel(x, edge_index, edge_attr, batch, params):
    N = x.shape[0]
    E = edge_index.shape[1]
    f32 = jnp.float32
    src = edge_index[0]
    dst = edge_index[1]
    ea8 = jnp.concatenate(
        [edge_attr, jnp.zeros((E, 1), f32)], axis=1)
    wee8 = jnp.concatenate(
        [params['ee'][0], jnp.zeros((1, H), f32)], axis=0)
    eeb = params['ee'][1][None, :]

    h = pl.pallas_call(
        _prologue_body,
        out_shape=jax.ShapeDtypeStruct((N, H), f32),
    )(x, params['ne'][0], params['ne'][1][None, :])

    ES = E // 2
    slabs = [(dst[:ES], src[:ES], ea8[:ES]), (dst[ES:], src[ES:], ea8[ES:])]
    gather = _make_gather(ES, N)
    scatter = _make_scatter(ES, N)
    nblk = ES // BE
    zeros_acc = jnp.zeros((N + 8, H), f32)

    for lp in params['layers']:
        a1w, a1b = lp['a1']
        m1w, m1b = lp['m1']
        wd = jnp.concatenate([a1w[:H], m1w[:H]], axis=1)
        ws = jnp.concatenate([a1w[H:2 * H], m1w[H:2 * H]], axis=1)
        we = jnp.concatenate([a1w[2 * H:], m1w[2 * H:]], axis=1)
        bcat = jnp.concatenate([a1b, m1b])[None, :]
        a2p = jnp.concatenate(
            [lp['a2'][0], jnp.zeros((H, 8 - HEADS), f32)], axis=1)
        a2bp = jnp.concatenate(
            [lp['a2'][1], jnp.zeros((8 - HEADS,), f32)])[:, None]
        m2w, m2b = lp['m2']
        u1w, u1b = lp['u1']

        gathered = [gather(h, d, s) for d, s, _ in slabs]

        rms, lts = [], []
        for (hd, hs), (_, _, eas) in zip(gathered, slabs):
            rm, lt = pl.pallas_call(
                _edge_body,
                grid=(nblk,),
                in_specs=[
                    pl.BlockSpec((BE, H), lambda i: (i, 0)),
                    pl.BlockSpec((BE, H), lambda i: (i, 0)),
                    pl.BlockSpec((BE, 8), lambda i: (i, 0)),
                    pl.BlockSpec((8, H), lambda i: (0, 0)),
                    pl.BlockSpec((1, H), lambda i: (0, 0)),
                    pl.BlockSpec((H, 2 * H), lambda i: (0, 0)),
                    pl.BlockSpec((H, 2 * H), lambda i: (0, 0)),
                    pl.BlockSpec((H, 2 * H), lambda i: (0, 0)),
                    pl.BlockSpec((1, 2 * H), lambda i: (0, 0)),
                    pl.BlockSpec((H, 8), lambda i: (0, 0)),
                    pl.BlockSpec((8, 1), lambda i: (0, 0)),
                ],
                out_specs=[
                    pl.BlockSpec((BE, H), lambda i: (i, 0)),
                    pl.BlockSpec((8, BE), lambda i: (0, i)),
                ],
                out_shape=[
                    jax.ShapeDtypeStruct((ES, H), f32),
                    jax.ShapeDtypeStruct((8, ES), f32),
                ],
            )(hd, hs, eas, wee8, eeb, wd, ws, we, bcat, a2p, a2bp)
            rms.append(rm)
            lts.append(lt)

        w = pl.pallas_call(
            _softmax_body,
            out_shape=jax.ShapeDtypeStruct((1, E), f32),
        )(lts[0], lts[1])
        w = w.reshape(E, 1)

        parts = []
        for k, ((d, _, _), rm) in enumerate(zip(slabs, rms)):
            wm = pl.pallas_call(
                _scale_body,
                grid=(nblk,),
                in_specs=[
                    pl.BlockSpec((BE, H), lambda i: (i, 0)),
                    pl.BlockSpec((BE, 1), lambda i: (i, 0)),
                    pl.BlockSpec((H, H), lambda i: (0, 0)),
                    pl.BlockSpec((1, H), lambda i: (0, 0)),
                ],
                out_specs=pl.BlockSpec((BE, H), lambda i: (i, 0)),
                out_shape=jax.ShapeDtypeStruct((ES, H), f32),
            )(rm, w[k * ES:(k + 1) * ES], m2w, m2b[None, :])
            parts.append(scatter(wm, d, zeros_acc))

        h = pl.pallas_call(
            _node_body,
            out_shape=jax.ShapeDtypeStruct((N, H), f32),
        )(parts[0], parts[1], h, u1w[:H], u1w[H:], u1b[None, :],
          lp['u2'][0], lp['u2'][1][None, :],
          lp['gn_w'][None, :], lp['gn_b'][None, :], lp['gn_ms'][None, :],
          lp['ln_w'][None, :], lp['ln_b'][None, :])

    hp = params['heads']
    w1 = jnp.concatenate([hp[k]['h1'][0] for k in ('tensile', 'tg', 'flex')], 0)
    b1 = jnp.stack([hp[k]['h1'][1] for k in ('tensile', 'tg', 'flex')], 0)
    w2 = jnp.concatenate([hp[k]['h2'][0] for k in ('tensile', 'tg', 'flex')], 0)
    b2 = jnp.stack([hp[k]['h2'][1] for k in ('tensile', 'tg', 'flex')], 0)
    w3 = jnp.concatenate(
        [jnp.concatenate([hp[k]['h3'][0],
                          jnp.zeros((64, 127), f32)], axis=1)
         for k in ('tensile', 'tg', 'flex')], 0)
    b3 = jnp.stack([jnp.concatenate([hp[k]['h3'][1],
                                     jnp.zeros((127,), f32)])
                    for k in ('tensile', 'tg', 'flex')], 0)

    outm = pl.pallas_call(
        _pool_body,
        out_shape=jax.ShapeDtypeStruct((NUM_GRAPHS, 4), f32),
        scratch_shapes=[pltpu.VMEM((NUM_GRAPHS, H), f32)],
    )(h, batch[:, None], w1, b1, w2, b2, w3, b3)

    return (outm[:, 0], outm[:, 1], outm[:, 2], outm[:, 3])
